# Initial kernel scaffold; baseline (speedup 1.0000x reference)
import jax
import jax.numpy as jnp
from jax.experimental import pallas as pl


def kernel(h, edge_index0, edge_index1, edge_index2, W0, al0, ar0, b0, W1, al1, ar1, b1, W2, al2, ar2, b2, sW1, sb1, sW2):
    raise NotImplementedError("write your pallas kernel here")


# speedup vs baseline: 84.5928x; 84.5928x over previous
import functools

import jax
import jax.numpy as jnp
from jax import lax
from jax.experimental import pallas as pl
from jax.experimental.pallas import tpu as pltpu
from jax.experimental.pallas import tpu_sc as plsc

N = 10000
E = 320000
IN_DIM = 128
OUT_DIM = 16
H = 8
M = 3
HID = 128
REC = 144
NC = 2
NS = 16
NW = NC * NS
EPT = E // NW
K = 80
NCHUNK = EPT // K
NPAD = 10240
ROWS_PT = NPAD // NS
NEG = -1.0e30

BA = 400
NBLK = N // BA


def _bcast_lane(v, h):
    idx = jnp.full((16, 1), h, dtype=jnp.int32)
    return lax.gather(
        v, idx,
        lax.GatherDimensionNumbers(
            offset_dims=(), collapsed_slice_dims=(0,), start_index_map=(0,)),
        (1,), mode=lax.GatherScatterMode.PROMISE_IN_BOUNDS)


def _prep_body(h_ref, w_ref, almat_ref, armat_ref, rec_ref, ert_ref):
    f = jnp.dot(h_ref[...], w_ref[0], preferred_element_type=jnp.float32)
    lanes = lax.broadcasted_iota(jnp.int32, (1, 16), 1)
    padv = jnp.where(lanes < 8, 0.0, NEG)
    el16 = jnp.dot(f, almat_ref[0], preferred_element_type=jnp.float32) + padv
    er16 = jnp.dot(f, armat_ref[0], preferred_element_type=jnp.float32) + padv
    rec_ref[:, 0:128] = f
    rec_ref[:, 128:144] = el16
    ert_ref[...] = er16


def _tc_prep(h, Ws, almat, armat):
    return pl.pallas_call(
        _prep_body,
        grid=(M, NBLK),
        in_specs=[
            pl.BlockSpec((BA, IN_DIM), lambda m, i: (i, 0)),
            pl.BlockSpec((1, IN_DIM, IN_DIM), lambda m, i: (m, 0, 0)),
            pl.BlockSpec((1, IN_DIM, 16), lambda m, i: (m, 0, 0)),
            pl.BlockSpec((1, IN_DIM, 16), lambda m, i: (m, 0, 0)),
        ],
        out_specs=[
            pl.BlockSpec((BA, REC), lambda m, i: (m * NBLK + i, 0)),
            pl.BlockSpec((BA, 16), lambda m, i: (m * NBLK + i, 0)),
        ],
        out_shape=[
            jax.ShapeDtypeStruct((M * N, REC), jnp.float32),
            jax.ShapeDtypeStruct((M * N, 16), jnp.float32),
        ],
    )(h, Ws, almat, armat)


def _sc_body(rec_hbm, ert_hbm, src_hbm, dst_hbm, out_hbm,
             acc, sidx, didx, didxo, srcbuf, erbuf, stage,
             gsem, esem):
    c = lax.axis_index("c")
    s = lax.axis_index("s")
    ebase0 = (c * NS + s) * EPT

    def _metapath(m, carry):
        def _zrow(r, cc):
            for j in range(REC // 16):
                stage[r, pl.ds(16 * j, 16)] = jnp.zeros((16,), jnp.float32)
            return cc
        lax.fori_loop(0, K, _zrow, 0)

        def _zacc(r, cc):
            pltpu.sync_copy(stage, acc.at[pl.ds(s * ROWS_PT + r * K, K)])
            return cc
        lax.fori_loop(0, ROWS_PT // K, _zacc, 0)
        plsc.subcore_barrier()

        moff = m * N
        ebase = m * E + ebase0

        def _chunk(g, cc):
            off = ebase + g * K
            pltpu.sync_copy(src_hbm.at[pl.ds(off, K)], sidx)
            pltpu.sync_copy(dst_hbm.at[pl.ds(off, K)], didx)
            for j in range(K // 16):
                sl = pl.ds(16 * j, 16)
                sidx[sl] = sidx[sl] + moff
                didxo[sl] = didx[sl] + moff
            cp1 = pltpu.async_copy(rec_hbm.at[sidx], srcbuf, gsem)
            cp2 = pltpu.async_copy(ert_hbm.at[didxo], erbuf, esem)
            cp1.wait()
            cp2.wait()
            for e in range(K):
                a = srcbuf[e, pl.ds(128, 16)]
                b = erbuf[e, :]
                sc = a + b
                sc = jnp.where(sc > 0, sc, sc * jnp.float32(0.2))
                ex = jnp.exp(sc)
                stage[e, pl.ds(128, 16)] = ex
                for hh in range(H):
                    fv = srcbuf[e, pl.ds(16 * hh, 16)]
                    stage[e, pl.ds(16 * hh, 16)] = fv * _bcast_lane(ex, hh)
            pltpu.sync_copy(stage, acc.at[didx], add=True)
            return cc
        lax.fori_loop(0, NCHUNK, _chunk, 0)

        plsc.subcore_barrier()
        rowoff = (m * NC + c) * NPAD + s * ROWS_PT
        pltpu.sync_copy(acc.at[pl.ds(s * ROWS_PT, ROWS_PT)],
                        out_hbm.at[pl.ds(rowoff, ROWS_PT)])
        plsc.subcore_barrier()
        return carry
    lax.fori_loop(0, M, _metapath, 0)


def _sc_edge(rec, ert, src_all, dst_all):
    mesh = plsc.VectorSubcoreMesh(core_axis_name="c", subcore_axis_name="s",
                                  num_cores=NC, num_subcores=NS)
    f = pl.kernel(
        _sc_body,
        out_type=jax.ShapeDtypeStruct((M * NC * NPAD, REC), jnp.float32),
        mesh=mesh,
        scratch_types=[
            pltpu.VMEM_SHARED((NPAD, REC), jnp.float32),
            pltpu.VMEM((K,), jnp.int32),
            pltpu.VMEM((K,), jnp.int32),
            pltpu.VMEM((K,), jnp.int32),
            pltpu.VMEM((K, REC), jnp.float32),
            pltpu.VMEM((K, 16), jnp.float32),
            pltpu.VMEM((K, REC), jnp.float32),
            pltpu.SemaphoreType.DMA,
            pltpu.SemaphoreType.DMA,
        ],
        compiler_params=pltpu.CompilerParams(use_tc_tiling_on_sc=False),
    )
    return f(rec, ert, src_all, dst_all)


def _post_body(accr_ref, b_ref, exp8_ref, sW1_ref, sb1_ref, sW2_ref,
               z_ref, wpart_ref):
    i = pl.program_id(1)
    a = accr_ref[0, 0] + accr_ref[0, 1]
    msg = a[:, 0:128]
    s8 = a[:, 128:136]
    den = jnp.dot(s8, exp8_ref[...], preferred_element_type=jnp.float32) + 1e-9
    z = msg / den + b_ref[0]
    z = jnp.where(z > 0, z, jnp.exp(z) - 1.0)
    z_ref[0] = z
    t = jnp.tanh(jnp.dot(z, sW1_ref[...], preferred_element_type=jnp.float32)
                 + sb1_ref[...])
    pv = jnp.sum(t * sW2_ref[...])

    @pl.when(i == 0)
    def _():
        wpart_ref[...] = jnp.zeros_like(wpart_ref)

    wpart_ref[...] += pv


def _tc_post(accr, b_all, exp8, sW1, sb1r, sW2r):
    return pl.pallas_call(
        _post_body,
        grid=(M, NBLK),
        in_specs=[
            pl.BlockSpec((1, NC, BA, REC), lambda m, i: (m, 0, i, 0)),
            pl.BlockSpec((1, 1, IN_DIM), lambda m, i: (m, 0, 0)),
            pl.BlockSpec((8, IN_DIM), lambda m, i: (0, 0)),
            pl.BlockSpec((HID, HID), lambda m, i: (0, 0)),
            pl.BlockSpec((1, HID), lambda m, i: (0, 0)),
            pl.BlockSpec((1, HID), lambda m, i: (0, 0)),
        ],
        out_specs=[
            pl.BlockSpec((1, BA, 128), lambda m, i: (m, i, 0)),
            pl.BlockSpec((1, 8, 128), lambda m, i: (m, 0, 0)),
        ],
        out_shape=[
            jax.ShapeDtypeStruct((M, N, 128), jnp.float32),
            jax.ShapeDtypeStruct((M, 8, 128), jnp.float32),
        ],
    )(accr, b_all, exp8, sW1, sb1r, sW2r)


def _comb_body(z_ref, wpart_ref, out_ref):
    w0 = wpart_ref[0, 0, 0] / N
    w1 = wpart_ref[1, 0, 0] / N
    w2 = wpart_ref[2, 0, 0] / N
    mx = jnp.maximum(w0, jnp.maximum(w1, w2))
    e0 = jnp.exp(w0 - mx)
    e1 = jnp.exp(w1 - mx)
    e2 = jnp.exp(w2 - mx)
    ssum = e0 + e1 + e2
    out_ref[...] = (e0 * z_ref[0] + e1 * z_ref[1] + e2 * z_ref[2]) / ssum


def _tc_combine(z, wpart):
    return pl.pallas_call(
        _comb_body,
        grid=(NBLK,),
        in_specs=[
            pl.BlockSpec((M, BA, 128), lambda i: (0, i, 0)),
            pl.BlockSpec((M, 8, 128), lambda i: (0, 0, 0)),
        ],
        out_specs=pl.BlockSpec((BA, 128), lambda i: (i, 0)),
        out_shape=jax.ShapeDtypeStruct((N, 128), jnp.float32),
    )(z, wpart)


def _attn_mat(a):
    m = jnp.kron(jnp.eye(8, dtype=jnp.float32), jnp.ones((16, 1), jnp.float32))
    m = m * a.reshape(128, 1)
    return jnp.pad(m, ((0, 0), (0, 8)))


def kernel(h, edge_index0, edge_index1, edge_index2,
           W0, al0, ar0, b0, W1, al1, ar1, b1, W2, al2, ar2, b2,
           sW1, sb1, sW2):
    h = h.astype(jnp.float32)
    Ws = jnp.stack([W0, W1, W2])
    almat = jnp.stack([_attn_mat(al0), _attn_mat(al1), _attn_mat(al2)])
    armat = jnp.stack([_attn_mat(ar0), _attn_mat(ar1), _attn_mat(ar2)])
    rec, ert = _tc_prep(h, Ws, almat, armat)

    src_all = jnp.concatenate([edge_index0[0], edge_index1[0], edge_index2[0]])
    dst_all = jnp.concatenate([edge_index0[1], edge_index1[1], edge_index2[1]])
    acc = _sc_edge(rec, ert, src_all, dst_all)
    accr = acc.reshape(M, NC, NPAD, REC)

    b_all = jnp.stack([b0, b1, b2]).reshape(M, 1, IN_DIM)
    exp8 = jnp.kron(jnp.eye(8, dtype=jnp.float32), jnp.ones((1, 16), jnp.float32))
    z, wpart = _tc_post(accr, b_all, exp8, sW1, sb1.reshape(1, HID),
                        sW2.reshape(1, HID))
    return _tc_combine(z, wpart)
